# Initial kernel scaffold; baseline (speedup 1.0000x reference)
import jax
import jax.numpy as jnp
from jax.experimental import pallas as pl


def kernel(embedding_table, input):
    raise NotImplementedError("write your pallas kernel here")


# speedup vs baseline: 1.9317x; 1.9317x over previous
import functools

import jax
import jax.numpy as jnp
from jax import lax
from jax.experimental import pallas as pl
from jax.experimental.pallas import tpu as pltpu
from jax.experimental.pallas import tpu_sc as plsc

VOCAB = 1000000
D = 32
B = 4096
L = 200
NC = 2
NS = 16
NW = NC * NS
BPW = B // NW
CH0, CH1 = 104, 96

_mesh = plsc.VectorSubcoreMesh(core_axis_name="c", subcore_axis_name="s")


@functools.partial(
    pl.kernel,
    mesh=_mesh,
    out_type=jax.ShapeDtypeStruct((B * D,), jnp.float32),
    scratch_types=[
        pltpu.VMEM((BPW * L,), jnp.int32),
        pltpu.VMEM((L, D), jnp.float32),
        pltpu.VMEM((BPW * D,), jnp.float32),
        pltpu.SemaphoreType.DMA,
    ],
    compiler_params=pltpu.CompilerParams(use_tc_tiling_on_sc=False),
)
def _emb_avg(table_hbm, idx_hbm, out_hbm, idx_v, buf_v, out_v, sem):
    wid = lax.axis_index("s") * NC + lax.axis_index("c")
    base = wid * (BPW * L)
    pltpu.sync_copy(idx_hbm.at[pl.ds(base, BPW * L)], idx_v)

    def body(b, carry):
        off = b * L
        c1 = pltpu.async_copy(
            table_hbm.at[idx_v.at[pl.ds(off, CH0)]],
            buf_v.at[pl.ds(0, CH0)], sem)
        c2 = pltpu.async_copy(
            table_hbm.at[idx_v.at[pl.ds(off + CH0, CH1)]],
            buf_v.at[pl.ds(CH0, CH1)], sem)
        c1.wait()
        c2.wait()
        accs = [jnp.zeros((16,), jnp.float32) for _ in range(8)]
        for j in range(L):
            k = (j % 4) * 2
            accs[k] = accs[k] + buf_v[j, 0:16]
            accs[k + 1] = accs[k + 1] + buf_v[j, 16:32]
        s0 = ((accs[0] + accs[2]) + (accs[4] + accs[6])) * (1.0 / L)
        s1 = ((accs[1] + accs[3]) + (accs[5] + accs[7])) * (1.0 / L)
        out_v[pl.ds(b * D, 16)] = s0
        out_v[pl.ds(b * D + 16, 16)] = s1
        return carry

    lax.fori_loop(0, BPW, body, 0)
    pltpu.sync_copy(out_v, out_hbm.at[pl.ds(wid * (BPW * D), BPW * D)])


def kernel(embedding_table, input):
    idx_flat = input.reshape(-1)
    out = _emb_avg(embedding_table, idx_flat)
    return out.reshape(B, 1, D)


# speedup vs baseline: 2.1086x; 1.0916x over previous
import functools

import jax
import jax.numpy as jnp
from jax import lax
from jax.experimental import pallas as pl
from jax.experimental.pallas import tpu as pltpu
from jax.experimental.pallas import tpu_sc as plsc

VOCAB = 1000000
D = 32
B = 4096
L = 200
NC = 2
NS = 16
NW = NC * NS
BPW = B // NW
CH0, CH1 = 104, 96
NBUF = 4

_mesh = plsc.VectorSubcoreMesh(core_axis_name="c", subcore_axis_name="s")


@functools.partial(
    pl.kernel,
    mesh=_mesh,
    out_type=jax.ShapeDtypeStruct((B * D,), jnp.float32),
    scratch_types=[
        pltpu.VMEM((BPW * L,), jnp.int32),
        [pltpu.VMEM((L, D), jnp.float32) for _ in range(NBUF)],
        pltpu.VMEM((BPW * D,), jnp.float32),
        [pltpu.SemaphoreType.DMA for _ in range(NBUF)],
    ],
    compiler_params=pltpu.CompilerParams(use_tc_tiling_on_sc=False),
)
def _emb_avg(table_hbm, idx_hbm, out_hbm, idx_v, bufs, out_v, sems):
    wid = lax.axis_index("s") * NC + lax.axis_index("c")
    base = wid * (BPW * L)
    pltpu.sync_copy(idx_hbm.at[pl.ds(base, BPW * L)], idx_v)

    def fire(row, buf, sem):
        off = row * L
        pltpu.async_copy(
            table_hbm.at[idx_v.at[pl.ds(off, CH0)]],
            buf.at[pl.ds(0, CH0)], sem)
        pltpu.async_copy(
            table_hbm.at[idx_v.at[pl.ds(off + CH0, CH1)]],
            buf.at[pl.ds(CH0, CH1)], sem)

    def drain(buf, sem):
        pltpu.make_async_copy(
            table_hbm.at[pl.ds(0, CH0)], buf.at[pl.ds(0, CH0)], sem).wait()
        pltpu.make_async_copy(
            table_hbm.at[pl.ds(0, CH1)], buf.at[pl.ds(CH0, CH1)], sem).wait()

    def reduce_store(row, buf):
        accs = [jnp.zeros((16,), jnp.float32) for _ in range(8)]
        for j in range(L):
            k = (j % 4) * 2
            accs[k] = accs[k] + buf[j, 0:16]
            accs[k + 1] = accs[k + 1] + buf[j, 16:32]
        r0 = ((accs[0] + accs[2]) + (accs[4] + accs[6])) * (1.0 / L)
        r1 = ((accs[1] + accs[3]) + (accs[5] + accs[7])) * (1.0 / L)
        out_v[pl.ds(row * D, 16)] = r0
        out_v[pl.ds(row * D + 16, 16)] = r1

    for s in range(NBUF):
        fire(s, bufs[s], sems[s])

    def body(g, carry):
        for s in range(NBUF):
            row = g * NBUF + s
            drain(bufs[s], sems[s])
            reduce_store(row, bufs[s])

            @pl.when(row + NBUF < BPW)
            def _():
                fire(row + NBUF, bufs[s], sems[s])

        return carry

    lax.fori_loop(0, BPW // NBUF, body, 0)
    pltpu.sync_copy(out_v, out_hbm.at[pl.ds(wid * (BPW * D), BPW * D)])


def kernel(embedding_table, input):
    idx_flat = input.reshape(-1)
    out = _emb_avg(embedding_table, idx_flat)
    return out.reshape(B, 1, D)
